# baseline (speedup 1.0000x reference)
import functools

import jax
import jax.numpy as jnp
from jax import lax
from jax.experimental import pallas as pl
from jax.experimental.pallas import tpu as pltpu
from jax.experimental.pallas import tpu_sc as plsc

N = 10000
D = 128
DA = 136
NC = 2
NS = 16
NW = NC * NS
C = 128
NP = 10112
SP = NP // NS


def _sc_segment_sum(table, src_w, dst_w):
    kc = src_w.shape[1]
    mesh = plsc.VectorSubcoreMesh(core_axis_name="c", subcore_axis_name="s")

    @functools.partial(
        pl.kernel,
        out_type=jax.ShapeDtypeStruct((NC, NP, DA), jnp.float32),
        mesh=mesh,
        compiler_params=pltpu.CompilerParams(use_tc_tiling_on_sc=False),
        scratch_types=[
            pltpu.VMEM((kc, C), jnp.int32),
            pltpu.VMEM((kc, C), jnp.int32),
            pltpu.VMEM((C, DA), jnp.float32),
            pltpu.VMEM_SHARED((NP, DA), jnp.float32),
            pltpu.SemaphoreType.DMA,
        ],
    )
    def seg_sum(table_hbm, src_hbm, dst_hbm, out_hbm,
                src_v, dst_v, rows_v, acc, sem):
        cid = lax.axis_index("c")
        sid = lax.axis_index("s")
        wid = cid * NS + sid

        zeros16 = jnp.zeros((16,), jnp.float32)

        def zrow(i, _):
            for j in range(8):
                rows_v[i, pl.ds(j * 16, 16)] = zeros16
            rows_v[i, pl.ds(DA - 16, 16)] = zeros16
            return 0

        lax.fori_loop(0, C, zrow, 0)
        base = sid * SP
        for r in range(SP // C):
            pltpu.sync_copy(rows_v, acc.at[pl.ds(base + r * C, C)])
        rem = SP % C
        if rem:
            pltpu.sync_copy(rows_v.at[pl.ds(0, rem)],
                            acc.at[pl.ds(base + (SP // C) * C, rem)])
        pltpu.sync_copy(src_hbm.at[wid], src_v)
        pltpu.sync_copy(dst_hbm.at[wid], dst_v)
        plsc.subcore_barrier()

        def chunk(k, _):
            pltpu.async_copy(table_hbm.at[src_v.at[k]], rows_v, sem).wait()
            pltpu.sync_copy(rows_v, acc.at[dst_v.at[k]], add=True)
            return 0

        lax.fori_loop(0, kc, chunk, 0)
        plsc.subcore_barrier()

        pltpu.sync_copy(acc.at[pl.ds(sid * SP, SP)],
                        out_hbm.at[cid, pl.ds(sid * SP, SP)])

    return seg_sum(table, src_w, dst_w)


def _combine_body(p_ref, fd_ref, w_ref, b_ref, a_ref, o_ref):
    x = p_ref[...]
    s = x[0] + x[1]
    agg = s[:, :D] + fd_ref[...]
    deg = s[:, D:D + 1] + 1.0
    y = jnp.dot(agg / deg, w_ref[...], preferred_element_type=jnp.float32)
    y = y + b_ref[...]
    a = a_ref[0, 0]
    o_ref[...] = jnp.where(y > 0, y, a * y)


def _tc_combine(parts, feat_dst, W, b, prelu_a):
    R = 1000
    grid = (N // R,)
    return pl.pallas_call(
        _combine_body,
        grid=grid,
        in_specs=[
            pl.BlockSpec((NC, R, DA), lambda i: (0, i, 0)),
            pl.BlockSpec((R, D), lambda i: (i, 0)),
            pl.BlockSpec((D, D), lambda i: (0, 0)),
            pl.BlockSpec((1, D), lambda i: (0, 0)),
            pl.BlockSpec((1, 1), lambda i: (0, 0)),
        ],
        out_specs=pl.BlockSpec((R, D), lambda i: (i, 0)),
        out_shape=jax.ShapeDtypeStruct((N, D), jnp.float32),
    )(parts, feat_dst, W, b.reshape(1, D), prelu_a.reshape(1, 1))


def kernel(feat, edge_index, feat_dst, W, b, prelu_a):
    E = edge_index.shape[1]
    ew = -(-E // NW)
    kc = -(-ew // C)
    ep = NW * kc * C

    src = edge_index[0]
    dst = edge_index[1]
    src_p = jnp.concatenate(
        [src, jnp.zeros((ep - E,), jnp.int32)]
    ).reshape(NW, kc, C)
    dst_p = jnp.concatenate(
        [dst, N + jnp.arange(ep - E, dtype=jnp.int32) % (NP - N)]
    ).reshape(NW, kc, C)

    ones_col = jnp.ones((N, 1), jnp.float32)
    zpad = jnp.zeros((N, DA - D - 1), jnp.float32)
    table = jnp.concatenate([feat, ones_col, zpad], axis=1)

    parts = _sc_segment_sum(table, src_p, dst_p)
    return _tc_combine(parts, feat_dst, W, b,
                       jnp.asarray(prelu_a, jnp.float32))


# speedup vs baseline: 1.6529x; 1.6529x over previous
import functools

import jax
import jax.numpy as jnp
from jax import lax
from jax.experimental import pallas as pl
from jax.experimental.pallas import tpu as pltpu
from jax.experimental.pallas import tpu_sc as plsc

N = 10000
D = 128
DA = 136
NC = 2
NS = 16
NW = NC * NS
C = 128
NP = 10112
SP = NP // NS


def _sc_segment_sum(table, src_w, dst_w):
    kc = src_w.shape[1]
    mesh = plsc.VectorSubcoreMesh(core_axis_name="c", subcore_axis_name="s")

    @functools.partial(
        pl.kernel,
        out_type=jax.ShapeDtypeStruct((NC, NP, DA), jnp.float32),
        mesh=mesh,
        compiler_params=pltpu.CompilerParams(use_tc_tiling_on_sc=False),
        scratch_types=[
            pltpu.VMEM((kc, C), jnp.int32),
            pltpu.VMEM((kc, C), jnp.int32),
            pltpu.VMEM((C, DA), jnp.float32),
            pltpu.VMEM_SHARED((NP, DA), jnp.float32),
            pltpu.SemaphoreType.DMA,
        ],
    )
    def seg_sum(table_hbm, src_hbm, dst_hbm, out_hbm,
                src_v, dst_v, rows_v, acc, sem):
        cid = lax.axis_index("c")
        sid = lax.axis_index("s")
        wid = cid * NS + sid

        zeros16 = jnp.zeros((16,), jnp.float32)

        def zrow(i, _):
            for j in range(8):
                rows_v[i, pl.ds(j * 16, 16)] = zeros16
            rows_v[i, pl.ds(DA - 16, 16)] = zeros16
            return 0

        lax.fori_loop(0, C, zrow, 0)
        base = sid * SP
        for r in range(SP // C):
            pltpu.sync_copy(rows_v, acc.at[pl.ds(base + r * C, C)])
        rem = SP % C
        if rem:
            pltpu.sync_copy(rows_v.at[pl.ds(0, rem)],
                            acc.at[pl.ds(base + (SP // C) * C, rem)])
        pltpu.sync_copy(src_hbm.at[wid], src_v)
        pltpu.sync_copy(dst_hbm.at[wid], dst_v)
        plsc.subcore_barrier()

        def chunk(k, _):
            pltpu.async_copy(table_hbm.at[src_v.at[k]], rows_v, sem).wait()
            pltpu.sync_copy(rows_v, acc.at[dst_v.at[k]], add=True)
            return 0

        lax.fori_loop(0, kc, chunk, 0)
        plsc.subcore_barrier()

        pltpu.sync_copy(acc.at[pl.ds(sid * SP, SP)],
                        out_hbm.at[cid, pl.ds(sid * SP, SP)])

    return seg_sum(table, src_w, dst_w)


def _combine_body(p_ref, fd_ref, w_ref, b_ref, a_ref, o_ref):
    x = p_ref[...]
    s = x[0] + x[1]
    agg = s[:, :D] + fd_ref[...]
    deg = s[:, D:D + 1] + 1.0
    y = jnp.dot(agg / deg, w_ref[...], preferred_element_type=jnp.float32)
    y = y + b_ref[...]
    a = a_ref[0, 0]
    o_ref[...] = jnp.where(y > 0, y, a * y)


def _tc_combine(parts, feat_dst, W, b, prelu_a):
    R = 1000
    grid = (N // R,)
    return pl.pallas_call(
        _combine_body,
        grid=grid,
        in_specs=[
            pl.BlockSpec((NC, R, DA), lambda i: (0, i, 0)),
            pl.BlockSpec((R, D), lambda i: (i, 0)),
            pl.BlockSpec((D, D), lambda i: (0, 0)),
            pl.BlockSpec((1, D), lambda i: (0, 0)),
            pl.BlockSpec((1, 1), lambda i: (0, 0)),
        ],
        out_specs=pl.BlockSpec((R, D), lambda i: (i, 0)),
        out_shape=jax.ShapeDtypeStruct((N, D), jnp.float32),
    )(parts, feat_dst, W, b.reshape(1, D), prelu_a.reshape(1, 1))


def kernel(feat, edge_index, feat_dst, W, b, prelu_a):
    E = edge_index.shape[1]
    ew = -(-E // NW)
    kc = -(-ew // C)
    ep = NW * kc * C

    src = edge_index[0]
    dst = edge_index[1]
    src_p = jnp.concatenate(
        [src, jnp.arange(ep - E, dtype=jnp.int32) % N]
    ).reshape(NW, kc, C)
    dst_p = jnp.concatenate(
        [dst, N + jnp.arange(ep - E, dtype=jnp.int32) % (NP - N)]
    ).reshape(NW, kc, C)

    ones_col = jnp.ones((N, 1), jnp.float32)
    zpad = jnp.zeros((N, DA - D - 1), jnp.float32)
    table = jnp.concatenate([feat, ones_col, zpad], axis=1)

    parts = _sc_segment_sum(table, src_p, dst_p)
    return _tc_combine(parts, feat_dst, W, b,
                       jnp.asarray(prelu_a, jnp.float32))


# speedup vs baseline: 1.9946x; 1.2067x over previous
import functools

import jax
import jax.numpy as jnp
from jax import lax
from jax.experimental import pallas as pl
from jax.experimental.pallas import tpu as pltpu
from jax.experimental.pallas import tpu_sc as plsc

N = 10000
D = 128
DG = 16
NC = 2
NS = 16
NW = NC * NS
C = 128
NP = 10112
SP = NP // NS


def _sc_segment_sum(feat, edge_index):
    E = edge_index.shape[1]
    ew = E // NW
    assert ew * NW == E
    kf = ew // C
    tail = ew - kf * C
    mesh = plsc.VectorSubcoreMesh(core_axis_name="c", subcore_axis_name="s")

    @functools.partial(
        pl.kernel,
        out_type=(
            jax.ShapeDtypeStruct((NC, NP, D), jnp.float32),
            jax.ShapeDtypeStruct((NC, NP, DG), jnp.float32),
        ),
        mesh=mesh,
        compiler_params=pltpu.CompilerParams(use_tc_tiling_on_sc=False),
        scratch_types=[
            pltpu.VMEM((ew,), jnp.int32),
            pltpu.VMEM((ew,), jnp.int32),
            pltpu.VMEM((C, D), jnp.float32),
            pltpu.VMEM((C, DG), jnp.float32),
            pltpu.VMEM_SHARED((NP, D), jnp.float32),
            pltpu.VMEM_SHARED((NP, DG), jnp.float32),
            pltpu.SemaphoreType.DMA,
        ],
    )
    def seg_sum(feat_hbm, edge_hbm, out_hbm, deg_hbm,
                src_v, dst_v, rows_v, ones_v, acc, dacc, sem):
        cid = lax.axis_index("c")
        sid = lax.axis_index("s")
        wid = cid * NS + sid

        zeros16 = jnp.zeros((16,), jnp.float32)

        def zrow(i, _):
            for j in range(D // 16):
                rows_v[i, pl.ds(j * 16, 16)] = zeros16
            ones_v[i, pl.ds(0, 16)] = zeros16
            return 0

        lax.fori_loop(0, C, zrow, 0)
        base = sid * SP
        for r in range(SP // C):
            pltpu.sync_copy(rows_v, acc.at[pl.ds(base + r * C, C)])
            pltpu.sync_copy(ones_v, dacc.at[pl.ds(base + r * C, C)])
        rem = SP % C
        if rem:
            pltpu.sync_copy(rows_v.at[pl.ds(0, rem)],
                            acc.at[pl.ds(base + (SP // C) * C, rem)])
            pltpu.sync_copy(ones_v.at[pl.ds(0, rem)],
                            dacc.at[pl.ds(base + (SP // C) * C, rem)])
        ones16 = jnp.ones((16,), jnp.float32)

        def orow(i, _):
            ones_v[i, pl.ds(0, 16)] = ones16
            return 0

        lax.fori_loop(0, C, orow, 0)
        pltpu.sync_copy(edge_hbm.at[0, pl.ds(wid * ew, ew)], src_v)
        pltpu.sync_copy(edge_hbm.at[1, pl.ds(wid * ew, ew)], dst_v)
        plsc.subcore_barrier()

        def chunk(k, _):
            pltpu.async_copy(feat_hbm.at[src_v.at[pl.ds(k * C, C)]],
                             rows_v, sem).wait()
            pltpu.sync_copy(rows_v, acc.at[dst_v.at[pl.ds(k * C, C)]],
                            add=True)
            pltpu.sync_copy(ones_v, dacc.at[dst_v.at[pl.ds(k * C, C)]],
                            add=True)
            return 0

        lax.fori_loop(0, kf, chunk, 0)
        if tail:
            pltpu.async_copy(feat_hbm.at[src_v.at[pl.ds(kf * C, tail)]],
                             rows_v.at[pl.ds(0, tail)], sem).wait()
            pltpu.sync_copy(rows_v.at[pl.ds(0, tail)],
                            acc.at[dst_v.at[pl.ds(kf * C, tail)]], add=True)
            pltpu.sync_copy(ones_v.at[pl.ds(0, tail)],
                            dacc.at[dst_v.at[pl.ds(kf * C, tail)]], add=True)
        plsc.subcore_barrier()

        pltpu.sync_copy(acc.at[pl.ds(sid * SP, SP)],
                        out_hbm.at[cid, pl.ds(sid * SP, SP)])
        pltpu.sync_copy(dacc.at[pl.ds(sid * SP, SP)],
                        deg_hbm.at[cid, pl.ds(sid * SP, SP)])

    return seg_sum(feat, edge_index)


def _combine_body(p_ref, dg_ref, fd_ref, w_ref, b_ref, a_ref, o_ref):
    x = p_ref[...]
    dgs = dg_ref[...]
    agg = x[0] + x[1] + fd_ref[...]
    deg = dgs[0, :, :1] + dgs[1, :, :1] + 1.0
    y = jnp.dot(agg / deg, w_ref[...], preferred_element_type=jnp.float32)
    y = y + b_ref[...]
    a = a_ref[0, 0]
    o_ref[...] = jnp.where(y > 0, y, a * y)


def _tc_combine(parts, degs, feat_dst, W, b, prelu_a):
    R = 1000
    grid = (N // R,)
    return pl.pallas_call(
        _combine_body,
        grid=grid,
        in_specs=[
            pl.BlockSpec((NC, R, D), lambda i: (0, i, 0)),
            pl.BlockSpec((NC, R, DG), lambda i: (0, i, 0)),
            pl.BlockSpec((R, D), lambda i: (i, 0)),
            pl.BlockSpec((D, D), lambda i: (0, 0)),
            pl.BlockSpec((1, D), lambda i: (0, 0)),
            pl.BlockSpec((1, 1), lambda i: (0, 0)),
        ],
        out_specs=pl.BlockSpec((R, D), lambda i: (i, 0)),
        out_shape=jax.ShapeDtypeStruct((N, D), jnp.float32),
    )(parts, degs, feat_dst, W, b.reshape(1, D), prelu_a.reshape(1, 1))


def kernel(feat, edge_index, feat_dst, W, b, prelu_a):
    parts, degs = _sc_segment_sum(feat, edge_index)
    return _tc_combine(parts, degs, feat_dst, W, b,
                       jnp.asarray(prelu_a, jnp.float32))


# speedup vs baseline: 2.0865x; 1.0461x over previous
import functools

import jax
import jax.numpy as jnp
from jax import lax
from jax.experimental import pallas as pl
from jax.experimental.pallas import tpu as pltpu
from jax.experimental.pallas import tpu_sc as plsc

N = 10000
D = 128
DG = 16
NC = 2
NS = 16
NW = NC * NS
C = 64
NP = 10112
SP = NP // NS


def _sc_segment_sum(feat, edge_index):
    E = edge_index.shape[1]
    ew = E // NW
    assert ew * NW == E
    kf = ew // C
    tail = ew - kf * C
    mesh = plsc.VectorSubcoreMesh(core_axis_name="c", subcore_axis_name="s")

    @functools.partial(
        pl.kernel,
        out_type=(
            jax.ShapeDtypeStruct((NC, NP, D), jnp.float32),
            jax.ShapeDtypeStruct((NC, NP, DG), jnp.float32),
        ),
        mesh=mesh,
        compiler_params=pltpu.CompilerParams(use_tc_tiling_on_sc=False),
        scratch_types=[
            pltpu.VMEM((ew,), jnp.int32),
            pltpu.VMEM((ew,), jnp.int32),
            pltpu.VMEM((2, C, D), jnp.float32),
            pltpu.VMEM((C, DG), jnp.float32),
            pltpu.VMEM_SHARED((NP, D), jnp.float32),
            pltpu.VMEM_SHARED((NP, DG), jnp.float32),
            pltpu.SemaphoreType.DMA,
        ],
    )
    def seg_sum(feat_hbm, edge_hbm, out_hbm, deg_hbm,
                src_v, dst_v, rows_v, ones_v, acc, dacc, sem):
        cid = lax.axis_index("c")
        sid = lax.axis_index("s")
        wid = cid * NS + sid

        zeros16 = jnp.zeros((16,), jnp.float32)

        def zrow(i, _):
            for j in range(D // 16):
                rows_v[0, i, pl.ds(j * 16, 16)] = zeros16
            ones_v[i, pl.ds(0, 16)] = zeros16
            return 0

        lax.fori_loop(0, C, zrow, 0)
        base = sid * SP
        for r in range(SP // C):
            pltpu.sync_copy(rows_v.at[0], acc.at[pl.ds(base + r * C, C)])
            pltpu.sync_copy(ones_v, dacc.at[pl.ds(base + r * C, C)])
        rem = SP % C
        if rem:
            pltpu.sync_copy(rows_v.at[0, pl.ds(0, rem)],
                            acc.at[pl.ds(base + (SP // C) * C, rem)])
            pltpu.sync_copy(ones_v.at[pl.ds(0, rem)],
                            dacc.at[pl.ds(base + (SP // C) * C, rem)])
        ones16 = jnp.ones((16,), jnp.float32)

        def orow(i, _):
            ones_v[i, pl.ds(0, 16)] = ones16
            return 0

        lax.fori_loop(0, C, orow, 0)
        pltpu.sync_copy(edge_hbm.at[0, pl.ds(wid * ew, ew)], src_v)
        pltpu.sync_copy(edge_hbm.at[1, pl.ds(wid * ew, ew)], dst_v)
        plsc.subcore_barrier()

        pltpu.async_copy(feat_hbm.at[src_v.at[pl.ds(0, C)]],
                         rows_v.at[0], sem)

        def chunk(k, _):
            buf = lax.rem(k, 2)
            pltpu.make_async_copy(feat_hbm.at[src_v.at[pl.ds(k * C, C)]],
                                  rows_v.at[buf], sem).wait()

            @pl.when(k + 1 < kf)
            def _next_gather():
                pltpu.async_copy(
                    feat_hbm.at[src_v.at[pl.ds((k + 1) * C, C)]],
                    rows_v.at[1 - buf], sem)

            pltpu.sync_copy(rows_v.at[buf],
                            acc.at[dst_v.at[pl.ds(k * C, C)]], add=True)
            pltpu.sync_copy(ones_v, dacc.at[dst_v.at[pl.ds(k * C, C)]],
                            add=True)
            return 0

        lax.fori_loop(0, kf, chunk, 0)
        if tail:
            pltpu.async_copy(feat_hbm.at[src_v.at[pl.ds(kf * C, tail)]],
                             rows_v.at[0, pl.ds(0, tail)], sem).wait()
            pltpu.sync_copy(rows_v.at[0, pl.ds(0, tail)],
                            acc.at[dst_v.at[pl.ds(kf * C, tail)]], add=True)
            pltpu.sync_copy(ones_v.at[pl.ds(0, tail)],
                            dacc.at[dst_v.at[pl.ds(kf * C, tail)]], add=True)
        plsc.subcore_barrier()

        pltpu.sync_copy(acc.at[pl.ds(sid * SP, SP)],
                        out_hbm.at[cid, pl.ds(sid * SP, SP)])
        pltpu.sync_copy(dacc.at[pl.ds(sid * SP, SP)],
                        deg_hbm.at[cid, pl.ds(sid * SP, SP)])

    return seg_sum(feat, edge_index)


def _combine_body(p_ref, dg_ref, fd_ref, w_ref, b_ref, a_ref, o_ref):
    x = p_ref[...]
    dgs = dg_ref[...]
    agg = x[0] + x[1] + fd_ref[...]
    deg = dgs[0, :, :1] + dgs[1, :, :1] + 1.0
    y = jnp.dot(agg / deg, w_ref[...], preferred_element_type=jnp.float32)
    y = y + b_ref[...]
    a = a_ref[0, 0]
    o_ref[...] = jnp.where(y > 0, y, a * y)


def _tc_combine(parts, degs, feat_dst, W, b, prelu_a):
    R = 1000
    grid = (N // R,)
    return pl.pallas_call(
        _combine_body,
        grid=grid,
        in_specs=[
            pl.BlockSpec((NC, R, D), lambda i: (0, i, 0)),
            pl.BlockSpec((NC, R, DG), lambda i: (0, i, 0)),
            pl.BlockSpec((R, D), lambda i: (i, 0)),
            pl.BlockSpec((D, D), lambda i: (0, 0)),
            pl.BlockSpec((1, D), lambda i: (0, 0)),
            pl.BlockSpec((1, 1), lambda i: (0, 0)),
        ],
        out_specs=pl.BlockSpec((R, D), lambda i: (i, 0)),
        out_shape=jax.ShapeDtypeStruct((N, D), jnp.float32),
    )(parts, degs, feat_dst, W, b.reshape(1, D), prelu_a.reshape(1, 1))


def kernel(feat, edge_index, feat_dst, W, b, prelu_a):
    parts, degs = _sc_segment_sum(feat, edge_index)
    return _tc_combine(parts, degs, feat_dst, W, b,
                       jnp.asarray(prelu_a, jnp.float32))
